# Initial kernel scaffold; baseline (speedup 1.0000x reference)
import jax
import jax.numpy as jnp
from jax.experimental import pallas as pl


def kernel(logits):
    raise NotImplementedError("write your pallas kernel here")


# speedup vs baseline: 1.0503x; 1.0503x over previous
import functools

import jax
import jax.numpy as jnp
from jax import lax
from jax.experimental import pallas as pl
from jax.experimental.pallas import tpu as pltpu

_K0 = 0
_K1 = 42
_KS2 = _K0 ^ _K1 ^ 0x1BD11BDA

_ROTS = ((13, 15, 26, 6), (17, 29, 16, 24))
_KS = (_K0, _K1, _KS2)

_TINY = float(jnp.finfo(jnp.float32).tiny)
_NEG_INF = float("-inf")


def _rotl(x, r):
    return (x << r) | lax.shift_right_logical(x, 32 - r)


def _threefry_bits(x1):
    x0 = x1
    x1 = _rotl(x1, 13)
    x1 = x0 ^ x1
    for r in _ROTS[0][1:]:
        x0 = x0 + x1
        x1 = _rotl(x1, r)
        x1 = x0 ^ x1
    x0 = x0 + jnp.int32(_KS[1])
    x1 = x1 + jnp.int32((_KS[2] + 1) & 0xFFFFFFFF)
    for i in range(1, 5):
        for r in _ROTS[i % 2]:
            x0 = x0 + x1
            x1 = _rotl(x1, r)
            x1 = x0 ^ x1
        x0 = x0 + jnp.int32(_KS[(i + 1) % 3])
        x1 = x1 + jnp.int32((_KS[(i + 2) % 3] + (i + 1)) & 0xFFFFFFFF)
    return x0 ^ x1


def _gumbel_from_bits(bits):
    float_bits = lax.shift_right_logical(bits, 9) | jnp.int32(0x3F800000)
    f = lax.bitcast_convert_type(float_bits, jnp.float32) - jnp.float32(1.0)
    u = jnp.maximum(jnp.float32(_TINY), f)
    return -jnp.log(-jnp.log(u))


def _sample_kernel(logits_ref, out_ref, best_val, best_col, *, vocab, block_w,
                   chunk_w, nblocks):
    j = pl.program_id(0)

    @pl.when(j == 0)
    def _init():
        best_val[...] = jnp.full_like(best_val, jnp.float32(_NEG_INF))
        best_col[...] = jnp.zeros_like(best_col)

    rows, _ = logits_ref.shape
    base = j * block_w
    row_iota = lax.broadcasted_iota(jnp.int32, (rows, chunk_w), 0)
    lane_iota = lax.broadcasted_iota(jnp.int32, (rows, chunk_w), 1)
    rowlane = row_iota * jnp.int32(vocab) + lane_iota + jnp.int32(_K1)

    def run_chunks(masked):
        val = best_val[...]
        col = best_col[...]
        for t in range(block_w // chunk_w):
            cbase = base + t * chunk_w
            g = _gumbel_from_bits(_threefry_bits(rowlane + cbase))
            score = logits_ref[:, t * chunk_w:(t + 1) * chunk_w] + g
            c = cbase + lane_iota
            if masked:
                score = jnp.where(c < vocab, score, jnp.float32(_NEG_INF))
            better = score > val
            val = jnp.where(better, score, val)
            col = jnp.where(better, c, col)
        best_val[...] = val
        best_col[...] = col

    @pl.when(j < nblocks - 1)
    def _main():
        run_chunks(masked=False)

    @pl.when(j == nblocks - 1)
    def _last():
        run_chunks(masked=True)
        v = best_val[...]
        cl = best_col[...]
        vmax = jnp.max(v, axis=1, keepdims=True)
        cmin = jnp.min(jnp.where(v == vmax, cl, jnp.int32(2**31 - 1)),
                       axis=1, keepdims=True)
        out_ref[...] = jnp.broadcast_to(cmin, out_ref.shape)


@functools.partial(jax.jit, static_argnames=("block_w", "chunk_w"))
def _sample(logits, block_w=2048, chunk_w=128):
    rows, vocab = logits.shape
    nblocks = pl.cdiv(vocab, block_w)
    out = pl.pallas_call(
        functools.partial(_sample_kernel, vocab=vocab, block_w=block_w,
                          chunk_w=chunk_w, nblocks=nblocks),
        grid=(nblocks,),
        in_specs=[pl.BlockSpec((rows, block_w), lambda j: (0, j))],
        out_specs=pl.BlockSpec((rows, chunk_w), lambda j: (0, 0)),
        out_shape=jax.ShapeDtypeStruct((rows, chunk_w), jnp.int32),
        scratch_shapes=[
            pltpu.VMEM((rows, chunk_w), jnp.float32),
            pltpu.VMEM((rows, chunk_w), jnp.int32),
        ],
        compiler_params=pltpu.CompilerParams(
            dimension_semantics=("arbitrary",)),
    )(logits)
    return out[:, :1]


def kernel(logits):
    return _sample(logits).astype(jnp.int64)


# speedup vs baseline: 1.0842x; 1.0322x over previous
import functools

import jax
import jax.numpy as jnp
from jax import lax
from jax.experimental import pallas as pl
from jax.experimental.pallas import tpu as pltpu

_K0 = 0
_K1 = 42
_KS2 = _K0 ^ _K1 ^ 0x1BD11BDA

_ROTS = ((13, 15, 26, 6), (17, 29, 16, 24))
_KS = (_K0, _K1, _KS2)

_TINY = float(jnp.finfo(jnp.float32).tiny)
_NEG_INF = float("-inf")


def _rotl(x, r):
    return (x << r) | lax.shift_right_logical(x, 32 - r)


def _threefry_bits(x1):
    x0 = x1
    x1 = _rotl(x1, 13)
    x1 = x0 ^ x1
    for r in _ROTS[0][1:]:
        x0 = x0 + x1
        x1 = _rotl(x1, r)
        x1 = x0 ^ x1
    x0 = x0 + jnp.int32(_KS[1])
    x1 = x1 + jnp.int32((_KS[2] + 1) & 0xFFFFFFFF)
    for i in range(1, 5):
        for r in _ROTS[i % 2]:
            x0 = x0 + x1
            x1 = _rotl(x1, r)
            x1 = x0 ^ x1
        x0 = x0 + jnp.int32(_KS[(i + 1) % 3])
        x1 = x1 + jnp.int32((_KS[(i + 2) % 3] + (i + 1)) & 0xFFFFFFFF)
    return x0 ^ x1


def _gumbel_from_bits(bits):
    float_bits = lax.shift_right_logical(bits, 9) | jnp.int32(0x3F800000)
    f = lax.bitcast_convert_type(float_bits, jnp.float32) - jnp.float32(1.0)
    u = jnp.maximum(jnp.float32(_TINY), f)
    return -jnp.log(-jnp.log(u))


def _sample_kernel(logits_ref, out_ref, best_val, best_col, *, vocab, block_w,
                   chunk_w, nblocks):
    j = pl.program_id(0)

    @pl.when(j == 0)
    def _init():
        best_val[...] = jnp.full_like(best_val, jnp.float32(_NEG_INF))
        best_col[...] = jnp.zeros_like(best_col)

    rows, _ = logits_ref.shape
    base = j * block_w
    row_iota = lax.broadcasted_iota(jnp.int32, (rows, chunk_w), 0)
    lane_iota = lax.broadcasted_iota(jnp.int32, (rows, chunk_w), 1)
    rowlane = row_iota * jnp.int32(vocab) + lane_iota + jnp.int32(_K1)

    def run_chunks(masked):
        val = best_val[...]
        col = best_col[...]
        for t in range(block_w // chunk_w):
            cbase = base + t * chunk_w
            g = _gumbel_from_bits(_threefry_bits(rowlane + cbase))
            score = logits_ref[:, t * chunk_w:(t + 1) * chunk_w] + g
            c = cbase + lane_iota
            if masked:
                score = jnp.where(c < vocab, score, jnp.float32(_NEG_INF))
            better = score > val
            val = jnp.where(better, score, val)
            col = jnp.where(better, c, col)
        best_val[...] = val
        best_col[...] = col

    @pl.when(j < nblocks - 1)
    def _main():
        run_chunks(masked=False)

    @pl.when(j == nblocks - 1)
    def _last():
        run_chunks(masked=True)
        v = best_val[...]
        cl = best_col[...]
        vmax = jnp.max(v, axis=1, keepdims=True)
        cmin = jnp.min(jnp.where(v == vmax, cl, jnp.int32(2**31 - 1)),
                       axis=1, keepdims=True)
        out_ref[...] = jnp.broadcast_to(cmin, out_ref.shape)


@functools.partial(jax.jit, static_argnames=("block_w", "chunk_w"))
def _sample(logits, block_w=4096, chunk_w=256):
    rows, vocab = logits.shape
    nblocks = pl.cdiv(vocab, block_w)
    out = pl.pallas_call(
        functools.partial(_sample_kernel, vocab=vocab, block_w=block_w,
                          chunk_w=chunk_w, nblocks=nblocks),
        grid=(nblocks,),
        in_specs=[pl.BlockSpec((rows, block_w), lambda j: (0, j))],
        out_specs=pl.BlockSpec((rows, chunk_w), lambda j: (0, 0)),
        out_shape=jax.ShapeDtypeStruct((rows, chunk_w), jnp.int32),
        scratch_shapes=[
            pltpu.VMEM((rows, chunk_w), jnp.float32),
            pltpu.VMEM((rows, chunk_w), jnp.int32),
        ],
        compiler_params=pltpu.CompilerParams(
            dimension_semantics=("arbitrary",)),
    )(logits)
    return out[:, :1]


def kernel(logits):
    return _sample(logits).astype(jnp.int64)
